# baseline (speedup 1.0000x reference)
import jax
import jax.numpy as jnp
from jax.experimental import pallas as pl
from jax.experimental.pallas import tpu as pltpu


def _dma_body(emb_ref, out_ref, sems):
    batch = out_ref.shape[0]
    seq_len = out_ref.shape[1]
    for b in range(batch):
        pltpu.make_async_copy(
            emb_ref.at[pl.ds(0, seq_len)], out_ref.at[b], sems.at[b]
        ).start()
    for b in range(batch):
        pltpu.make_async_copy(
            emb_ref.at[pl.ds(0, seq_len)], out_ref.at[b], sems.at[b]
        ).wait()


def kernel(x, pos_embedding):
    batch, seq_len = x.shape
    max_len, d_model = pos_embedding.shape

    out = pl.pallas_call(
        _dma_body,
        in_specs=[pl.BlockSpec(memory_space=pl.ANY)],
        out_specs=pl.BlockSpec(memory_space=pl.ANY),
        out_shape=jax.ShapeDtypeStruct((batch, seq_len, d_model),
                                       pos_embedding.dtype),
        scratch_shapes=[pltpu.SemaphoreType.DMA((batch,))],
    )(pos_embedding)
    return out


# speedup vs baseline: 81.2127x; 81.2127x over previous
import jax
import jax.numpy as jnp
from jax.experimental import pallas as pl
from jax.experimental.pallas import tpu as pltpu

_CHUNKS = 8


def _staged_body(emb_ref, out_ref, buf, rsem, wsem):
    batch = out_ref.shape[0]
    seq_len = out_ref.shape[1]
    rows = seq_len // _CHUNKS

    reads = []
    for i in range(_CHUNKS):
        c = pltpu.make_async_copy(
            emb_ref.at[pl.ds(i * rows, rows)],
            buf.at[pl.ds(i * rows, rows)],
            rsem.at[i],
        )
        c.start()
        reads.append(c)

    writes = []
    for i in range(_CHUNKS):
        reads[i].wait()
        for b in range(batch):
            c = pltpu.make_async_copy(
                buf.at[pl.ds(i * rows, rows)],
                out_ref.at[b, pl.ds(i * rows, rows)],
                wsem.at[i, b],
            )
            c.start()
            writes.append(c)

    for c in writes:
        c.wait()


def kernel(x, pos_embedding):
    batch, seq_len = x.shape
    max_len, d_model = pos_embedding.shape

    out = pl.pallas_call(
        _staged_body,
        in_specs=[pl.BlockSpec(memory_space=pl.ANY)],
        out_specs=pl.BlockSpec(memory_space=pl.ANY),
        out_shape=jax.ShapeDtypeStruct((batch, seq_len, d_model),
                                       pos_embedding.dtype),
        scratch_shapes=[
            pltpu.VMEM((seq_len, d_model), jnp.float32),
            pltpu.SemaphoreType.DMA((_CHUNKS,)),
            pltpu.SemaphoreType.DMA((_CHUNKS, 4)),
        ],
    )(pos_embedding)
    return out
